# Initial kernel scaffold; baseline (speedup 1.0000x reference)
import jax
import jax.numpy as jnp
from jax.experimental import pallas as pl


def kernel(x, W):
    raise NotImplementedError("write your pallas kernel here")


# speedup vs baseline: 1.5790x; 1.5790x over previous
import functools

import jax
import jax.numpy as jnp
from jax import lax
from jax.experimental import pallas as pl
from jax.experimental.pallas import tpu as pltpu
from jax.experimental.pallas import tpu_sc as plsc

NC = 2
NS = 16
NW = NC * NS
LANES = 16
CHUNK = 128
GROUPS = CHUNK // LANES
NBUF = 8
PREP_G = 128


def _prep_kernel(x_ref, idxm_ref, flags_ref):
  x = x_ref[...]
  idxm_ref[...] = jnp.maximum(x - 1, 0)
  flags_ref[...] = jnp.any(x == 0, axis=-1).astype(jnp.int32)[:, None, :]


def _emb_kernel(n_chunks, D, idxm_hbm, flags_hbm, x_hbm, w_hbm, out_hbm,
                idxm_v, rows_v, flags_v, xs_v, gsem, ssem):
  wid = lax.axis_index("s") * NC + lax.axis_index("c")
  base = wid * n_chunks * CHUNK

  pltpu.sync_copy(idxm_hbm.at[wid], idxm_v)
  pltpu.sync_copy(flags_hbm.at[wid], flags_v.at[pl.ds(0, n_chunks)])

  zv = jnp.zeros((LANES,), jnp.float32)

  def gather(g, b):
    return pltpu.make_async_copy(
        w_hbm.at[idxm_v.at[g]], rows_v.at[b], gsem.at[b])

  def store(g, b):
    return pltpu.make_async_copy(
        rows_v.at[b], out_hbm.at[pl.ds(base + g * CHUNK, CHUNK)],
        ssem.at[b])

  def fixup(g, b, flag):
    @pl.when(flag != 0)
    def _():
      pltpu.sync_copy(x_hbm.at[wid, g], xs_v)

      def grp(c, carry):
        v = xs_v[pl.ds(c * LANES, LANES)]
        for l in range(LANES):
          @pl.when(v[l] == 0)
          def _():
            for h in range(D // LANES):
              rows_v[b, c * LANES + l, pl.ds(h * LANES, LANES)] = zv
        return carry

      lax.fori_loop(0, GROUPS, grp, 0)

  for b in range(NBUF):
    gather(b, b).start()

  n_rounds = n_chunks // NBUF

  def round_body(rr, carry):
    fvec = flags_v[pl.ds(rr * NBUF, LANES)]
    for b in range(NBUF):
      g = rr * NBUF + b
      gather(g, b).wait()
      fixup(g, b, fvec[b])
      st = store(g, b)
      st.start()

      @pl.when(rr < n_rounds - 1)
      def _():
        st.wait()
        gather(g + NBUF, b).start()
    return carry

  lax.fori_loop(0, n_rounds, round_body, 0, unroll=1)

  for b in range(NBUF):
    store((n_rounds - 1) * NBUF + b, b).wait()


def kernel(x, W):
  B, L = x.shape
  V, D = W.shape
  N = B * L
  assert N % (NW * CHUNK) == 0
  n_chunks = N // (NW * CHUNK)
  assert n_chunks % NBUF == 0
  n_flat = N // CHUNK
  assert n_flat % PREP_G == 0
  n_blk = n_flat // PREP_G

  x3 = x.reshape(n_blk, PREP_G, CHUNK)
  idxm3, flags3 = pl.pallas_call(
      _prep_kernel,
      grid=(n_blk,),
      in_specs=[pl.BlockSpec((1, PREP_G, CHUNK), lambda i: (i, 0, 0))],
      out_specs=[
          pl.BlockSpec((1, PREP_G, CHUNK), lambda i: (i, 0, 0)),
          pl.BlockSpec((1, 1, PREP_G), lambda i: (i, 0, 0)),
      ],
      out_shape=[
          jax.ShapeDtypeStruct((n_blk, PREP_G, CHUNK), jnp.int32),
          jax.ShapeDtypeStruct((n_blk, 1, PREP_G), jnp.int32),
      ],
  )(x3)

  idxm = idxm3.reshape(NW, n_chunks, CHUNK)
  flags = flags3.reshape(NW, n_chunks)
  xr = x.reshape(NW, n_chunks, CHUNK)

  mesh = plsc.VectorSubcoreMesh(core_axis_name="c", subcore_axis_name="s")
  run = pl.kernel(
      functools.partial(_emb_kernel, n_chunks, D),
      out_type=jax.ShapeDtypeStruct((N, D), jnp.float32),
      mesh=mesh,
      compiler_params=pltpu.CompilerParams(use_tc_tiling_on_sc=False),
      scratch_types=[
          pltpu.VMEM((n_chunks, CHUNK), jnp.int32),
          pltpu.VMEM((NBUF, CHUNK, D), jnp.float32),
          pltpu.VMEM((n_chunks + LANES,), jnp.int32),
          pltpu.VMEM((CHUNK,), jnp.int32),
          pltpu.SemaphoreType.DMA((NBUF,)),
          pltpu.SemaphoreType.DMA((NBUF,)),
      ],
  )
  out = run(idxm, flags, xr, W)
  return out.reshape(B, L, D)
